# baseline (speedup 1.0000x reference)
import functools

import jax
import jax.numpy as jnp
from jax import lax
from jax.experimental import pallas as pl
from jax.experimental.pallas import tpu as pltpu
from jax.experimental.pallas import tpu_sc as plsc

N = 10000
E = 320000
F = 128
FE = 16
G = 64
C = 128
H = 128
NCLS = 4

SC_CORES = 2
SC_SUBCORES = 16
NW = SC_CORES * SC_SUBCORES
KC = 80
NCT = 128
EPW = NCT * KC
EPAD = NW * EPW
NPAD = 10240
RPS = NPAD // SC_SUBCORES
LANES = 16
CV = C // LANES


def _mp_sc(x, e, src3, dst3, eidx3):
    mesh = plsc.VectorSubcoreMesh(core_axis_name="c", subcore_axis_name="s")

    @functools.partial(
        pl.kernel,
        out_type=jax.ShapeDtypeStruct((SC_CORES, NPAD, C), jnp.float32),
        mesh=mesh,
        scratch_types=[
            [pltpu.VMEM((KC,), jnp.int32) for _ in range(8)],
            [pltpu.VMEM((KC,), jnp.int32) for _ in range(8)],
            [pltpu.VMEM((KC,), jnp.int32) for _ in range(8)],
            [pltpu.VMEM((KC, C), jnp.float32) for _ in range(4)],
            pltpu.VMEM_SHARED((NPAD, C), jnp.float32),
            [pltpu.SemaphoreType.DMA for _ in range(8)],
            [pltpu.SemaphoreType.DMA for _ in range(4)],
            [pltpu.SemaphoreType.DMA for _ in range(2)],
            [pltpu.SemaphoreType.DMA for _ in range(4)],
        ],
    )
    def mp(x_hbm, e_hbm, src3_hbm, dst3_hbm, eidx3_hbm, out_hbm,
           sv, dv, ev, mv, acc_sh, semi, semg, seme, sems):
        cid = lax.axis_index("c")
        sid = lax.axis_index("s")
        wid = sid * SC_CORES + cid

        zero = jnp.zeros((LANES,), jnp.float32)

        def zrow(r, carry):
            for cc in range(CV):
                mv[0][r, pl.ds(cc * LANES, LANES)] = zero
            return carry

        lax.fori_loop(0, KC, zrow, 0)
        for t in range(RPS // KC):
            pltpu.sync_copy(mv[0], acc_sh.at[pl.ds(sid * RPS + t * KC, KC)])
        plsc.subcore_barrier()

        def issue_idx(j, i8):
            pltpu.async_copy(src3_hbm.at[wid, j], sv[i8], semi[i8])
            pltpu.async_copy(dst3_hbm.at[wid, j], dv[i8], semi[i8])
            pltpu.async_copy(eidx3_hbm.at[wid, j], ev[i8], semi[i8])

        def issue_gx(j, i8, m4):
            pltpu.make_async_copy(src3_hbm.at[wid, j], sv[i8], semi[i8]).wait()
            pltpu.make_async_copy(dst3_hbm.at[wid, j], dv[i8], semi[i8]).wait()
            pltpu.async_copy(x_hbm.at[sv[i8]], mv[m4], semg[m4])

        def wait_gx(i8, m4):
            pltpu.make_async_copy(x_hbm.at[sv[i8]], mv[m4], semg[m4]).wait()

        def issue_eadd(j, i8, m4, e2):
            pltpu.make_async_copy(eidx3_hbm.at[wid, j], ev[i8],
                                  semi[i8]).wait()
            pltpu.async_copy(e_hbm.at[ev[i8]], mv[m4], seme[e2], add=True)

        def wait_eadd(i8, m4, e2):
            pltpu.make_async_copy(e_hbm.at[ev[i8]], mv[m4], seme[e2]).wait()

        def relu(m4):
            mb = mv[m4]

            def c4(it, carry):
                r = it * 4
                for dr in range(4):
                    for cc in range(CV):
                        sl = pl.ds(cc * LANES, LANES)
                        mb[r + dr, sl] = jnp.maximum(mb[r + dr, sl], 0.0)
                return carry

            lax.fori_loop(0, KC // 4, c4, 0)

        def issue_scatter(i8, m4):
            pltpu.async_copy(mv[m4], acc_sh.at[dv[i8]], sems[m4], add=True)

        def wait_scatter(i8, m4):
            pltpu.make_async_copy(mv[m4], acc_sh.at[dv[i8]], sems[m4]).wait()

        for b in range(4):
            issue_idx(b, b)
        issue_gx(0, 0, 0)
        issue_gx(1, 1, 1)
        wait_gx(0, 0)
        issue_eadd(0, 0, 0, 0)

        QN = NCT // 8

        def octet(q, carry):
            for b in range(8):
                j = 8 * q + b
                m4 = b % 4

                def wsc():
                    wait_scatter((b + 6) % 8, (b + 2) % 4)
                if b >= 2:
                    wsc()
                else:
                    @pl.when(q > 0)
                    def _():
                        wsc()

                def igx():
                    issue_gx(j + 2, (b + 2) % 8, (b + 2) % 4)
                if b < 6:
                    igx()
                else:
                    @pl.when(q < QN - 1)
                    def _():
                        igx()

                def ieadd():
                    wait_gx((b + 1) % 8, (b + 1) % 4)
                    issue_eadd(j + 1, (b + 1) % 8, (b + 1) % 4, (b + 1) % 2)
                if b < 7:
                    ieadd()
                else:
                    @pl.when(q < QN - 1)
                    def _():
                        ieadd()

                wait_eadd(b, m4, b % 2)
                relu(m4)
                issue_scatter(b, m4)

                def iidx():
                    issue_idx(j + 4, (b + 4) % 8)
                if b < 4:
                    iidx()
                else:
                    @pl.when(q < QN - 1)
                    def _():
                        iidx()
            return carry

        lax.fori_loop(0, QN, octet, 0)
        wait_scatter(6, 2)
        wait_scatter(7, 3)
        plsc.subcore_barrier()
        pltpu.sync_copy(acc_sh.at[pl.ds(sid * RPS, RPS)],
                        out_hbm.at[cid, pl.ds(sid * RPS, RPS)])

    return mp(x, e, src3, dst3, eidx3)


def _edge_embed(edge_attr, We0, be0, We1, be1):
    BE = 2048

    def body(ea_ref, w0_ref, b0_ref, w1_ref, b1_ref, e0_ref, e1_ref):
        ea = ea_ref[...]
        e0_ref[...] = jnp.dot(ea, w0_ref[...],
                              preferred_element_type=jnp.float32) + b0_ref[...]
        e1_ref[...] = jnp.dot(ea, w1_ref[...],
                              preferred_element_type=jnp.float32) + b1_ref[...]

    return pl.pallas_call(
        body,
        grid=(EPAD // BE,),
        in_specs=[
            pl.BlockSpec((BE, FE), lambda i: (i, 0)),
            pl.BlockSpec((FE, C), lambda i: (0, 0)),
            pl.BlockSpec((1, C), lambda i: (0, 0)),
            pl.BlockSpec((FE, C), lambda i: (0, 0)),
            pl.BlockSpec((1, C), lambda i: (0, 0)),
        ],
        out_specs=[
            pl.BlockSpec((BE, C), lambda i: (i, 0)),
            pl.BlockSpec((BE, C), lambda i: (i, 0)),
        ],
        out_shape=[
            jax.ShapeDtypeStruct((EPAD, C), jnp.float32),
            jax.ShapeDtypeStruct((EPAD, C), jnp.float32),
        ],
    )(edge_attr, We0, be0, We1, be1)


def _post(xin, part, Wn, bn, g, bt, scale):
    BN = 2000

    def body(s_ref, x_ref, p_ref, w_ref, b_ref, g_ref, t_ref, o_ref):
        h = s_ref[0] * x_ref[...] + p_ref[0] + p_ref[1]
        hh = jnp.dot(h, w_ref[...], preferred_element_type=jnp.float32) + b_ref[...]
        mu = jnp.mean(hh, axis=-1, keepdims=True)
        d = hh - mu
        var = jnp.mean(d * d, axis=-1, keepdims=True)
        y = d * lax.rsqrt(var + 1e-5) * g_ref[...] + t_ref[...]
        o_ref[...] = jnp.where(y > 0, y, 0.01 * y)

    return pl.pallas_call(
        body,
        grid=(N // BN,),
        in_specs=[
            pl.BlockSpec(memory_space=pltpu.SMEM),
            pl.BlockSpec((BN, C), lambda i: (i, 0)),
            pl.BlockSpec((SC_CORES, BN, C), lambda i: (0, i, 0)),
            pl.BlockSpec((C, C), lambda i: (0, 0)),
            pl.BlockSpec((1, C), lambda i: (0, 0)),
            pl.BlockSpec((1, C), lambda i: (0, 0)),
            pl.BlockSpec((1, C), lambda i: (0, 0)),
        ],
        out_specs=pl.BlockSpec((BN, C), lambda i: (i, 0)),
        out_shape=jax.ShapeDtypeStruct((N, C), jnp.float32),
    )(scale, xin, part, Wn, bn, g, bt)


def _pool(h2, batchf):
    def body(h_ref, b_ref, o_ref):
        gids = lax.broadcasted_iota(jnp.int32, (N, G), 1).astype(jnp.float32)
        onehot = (b_ref[...] == gids).astype(jnp.float32)
        o_ref[...] = lax.dot_general(
            onehot, h_ref[...], (((0,), (0,)), ((), ())),
            preferred_element_type=jnp.float32)

    return pl.pallas_call(
        body,
        in_specs=[
            pl.BlockSpec((N, C), lambda: (0, 0)),
            pl.BlockSpec((N, 1), lambda: (0, 0)),
        ],
        out_specs=pl.BlockSpec((G, C), lambda: (0, 0)),
        out_shape=jax.ShapeDtypeStruct((G, C), jnp.float32),
    )(h2, batchf)


def _classifier(h1, h2, h_pool, batchf, Wc, bc, Wf, bf):
    BN = 2000

    def body(h1_ref, h2_ref, hp_ref, b_ref, wc_ref, bc_ref, wf_ref, bf_ref,
             o_ref):
        gids = lax.broadcasted_iota(jnp.int32, (BN, G), 1).astype(jnp.float32)
        onehot = (b_ref[...] == gids).astype(jnp.float32)
        hp = jnp.dot(onehot, hp_ref[...], preferred_element_type=jnp.float32)
        wc = wc_ref[...]
        y = (jnp.dot(h1_ref[...], wc[0:C], preferred_element_type=jnp.float32)
             + jnp.dot(h2_ref[...], wc[C:2 * C],
                       preferred_element_type=jnp.float32)
             + jnp.dot(hp, wc[2 * C:3 * C], preferred_element_type=jnp.float32)
             + bc_ref[...])
        y = jnp.where(y > 0, y, 0.01 * y)
        z = jnp.dot(y, wf_ref[...], preferred_element_type=jnp.float32) + bf_ref[...]
        z = z - jnp.max(z, axis=-1, keepdims=True)
        ez = jnp.exp(z)
        o_ref[...] = ez / jnp.sum(ez, axis=-1, keepdims=True)

    return pl.pallas_call(
        body,
        grid=(N // BN,),
        in_specs=[
            pl.BlockSpec((BN, C), lambda i: (i, 0)),
            pl.BlockSpec((BN, C), lambda i: (i, 0)),
            pl.BlockSpec((G, C), lambda i: (0, 0)),
            pl.BlockSpec((BN, 1), lambda i: (i, 0)),
            pl.BlockSpec((3 * C, H), lambda i: (0, 0)),
            pl.BlockSpec((1, H), lambda i: (0, 0)),
            pl.BlockSpec((H, NCLS), lambda i: (0, 0)),
            pl.BlockSpec((1, NCLS), lambda i: (0, 0)),
        ],
        out_specs=pl.BlockSpec((BN, NCLS), lambda i: (i, 0)),
        out_shape=jax.ShapeDtypeStruct((N, NCLS), jnp.float32),
    )(h1, h2, h_pool, batchf, Wc, bc, Wf, bf)


def kernel(x, edge_index, edge_attr, batch,
           We0, be0, eps0, Wn0, bn0, g0, bt0,
           We1, be1, eps1, Wn1, bn1, g1, bt1,
           Wc, bc, Wf, bf):
    pad = EPAD - E
    src3 = jnp.concatenate(
        [edge_index[0], jnp.zeros((pad,), jnp.int32)]).reshape(NW, NCT, KC)
    dst3 = jnp.concatenate(
        [edge_index[1], jnp.full((pad,), N, jnp.int32)]).reshape(NW, NCT, KC)
    ea_p = jnp.concatenate([edge_attr, jnp.zeros((pad, FE), jnp.float32)])
    eidx3 = jnp.arange(EPAD, dtype=jnp.int32).reshape(NW, NCT, KC)
    batchf = batch.astype(jnp.float32).reshape(N, 1)
    r = lambda v: v.reshape(1, -1)

    e0, e1 = _edge_embed(ea_p, We0, r(be0), We1, r(be1))

    part0 = _mp_sc(x, e0, src3, dst3, eidx3)
    h1 = _post(x, part0, Wn0, r(bn0), r(g0), r(bt0), (1.0 + eps0).reshape(1))

    part1 = _mp_sc(h1, e1, src3, dst3, eidx3)
    h2 = _post(h1, part1, Wn1, r(bn1), r(g1), r(bt1), (1.0 + eps1).reshape(1))

    h_pool = _pool(h2, batchf)
    return _classifier(h1, h2, h_pool, batchf, Wc, r(bc), Wf, r(bf))
